# baseline (speedup 1.0000x reference)
import functools

import jax
import jax.numpy as jnp
from jax import lax
from jax.experimental import pallas as pl
from jax.experimental.pallas import tpu as pltpu
from jax.experimental.pallas import tpu_sc as plsc

K = 20
N = 2048
B_ROWS = 32768
NC, NS, L = 2, 16, 16
NW = NC * NS
ROWS_PER_W = B_ROWS // NW
CHUNK_ROWS = 8
CHUNKS_PER_W = ROWS_PER_W // CHUNK_ROWS
CHUNK_ELEMS = CHUNK_ROWS * N
OUT_PER_W = ROWS_PER_W * K

_INF = float("inf")


def _lane():
    return lax.iota(jnp.int32, L)


def _exact_sort16(v, i):
    w, j = plsc.sort_key_val(v, i)
    occ, _ = plsc.scan_count(w)
    rs = _lane() - occ
    key2 = rs * jnp.int32(2048) + j
    _, j2 = plsc.sort_key_val(key2, j)
    return w, j2


def _vsort(v):
    w, _ = plsc.sort_key_val(v, _lane())
    return w


def _lex_take(av, ai, bv, bi):
    return (av < bv) | ((av == bv) & (ai < bi))


def _merge32(a, b):
    rb = lax.rev(b, (0,))
    lo = jnp.minimum(a, rb)
    hi = jnp.maximum(a, rb)
    return _vsort(lo), _vsort(hi)


def _row_threshold(xb, base):
    inf16 = jnp.full((L,), _INF)
    def ph1(i, acc):
        a0, a1, a2, a3 = acc
        o = base + 64 * i
        a0 = jnp.minimum(a0, xb[pl.ds(o, L)])
        a1 = jnp.minimum(a1, xb[pl.ds(o + 16, L)])
        a2 = jnp.minimum(a2, xb[pl.ds(o + 32, L)])
        a3 = jnp.minimum(a3, xb[pl.ds(o + 48, L)])
        return a0, a1, a2, a3
    a0, a1, a2, a3 = lax.fori_loop(0, 32, ph1, (inf16, inf16, inf16, inf16))
    p0, p1 = _merge32(_vsort(a0), _vsort(a1))
    q0, q1 = _merge32(_vsort(a2), _vsort(a3))
    l0 = jnp.minimum(p0, lax.rev(q1, (0,)))
    l1 = jnp.minimum(p1, lax.rev(q0, (0,)))
    h = _vsort(jnp.maximum(l0, l1))
    t = jnp.min(jnp.where(_lane() >= 4, h, _INF))
    return t


def _row_body(xb, cand, ob, r, rw):
    base = N * r
    t = _row_threshold(xb, base)
    tv = jnp.full((L,), t)

    def ph2(i, carry):
        c, idxv = carry
        v = xb[pl.ds(base + 16 * i, L)]
        m = v <= tv
        plsc.store_compressed(cand.at[pl.ds(c, L)], idxv, mask=m)
        pc = plsc.all_reduce_population_count(m)
        c = c + lax.squeeze(lax.slice(pc, (0,), (1,)), (0,))
        return c, idxv + jnp.int32(16)
    c, _ = lax.fori_loop(0, 128, ph2, (jnp.int32(0), _lane()))

    inf16 = jnp.full((L,), _INF)
    zero16 = jnp.zeros((L,), jnp.int32)
    nv = (c + jnp.int32(15)) >> 4

    def ph3(jv, carry):
        sv0, sv1, si0, si1 = carry
        lanepos = _lane() + 16 * jv
        m = lanepos < c
        iraw = cand[pl.ds(16 * jv, L)]
        idx = jnp.where(m, iraw, 0)
        vals = plsc.load_gather(xb.at[pl.ds(base, N)], [idx])
        vals = jnp.where(m, vals, _INF)
        w, j = _exact_sort16(vals, idx)
        rw_, rj_ = lax.rev(w, (0,)), lax.rev(j, (0,))
        take = _lex_take(sv1, si1, rw_, rj_)
        l1v = jnp.where(take, sv1, rw_)
        l1i = jnp.where(take, si1, rj_)
        take2 = _lex_take(sv0, si0, l1v, l1i)
        lv = jnp.where(take2, sv0, l1v)
        li = jnp.where(take2, si0, l1i)
        hv = jnp.where(take2, l1v, sv0)
        hi = jnp.where(take2, l1i, si0)
        sv0, si0 = _exact_sort16(lv, li)
        sv1, si1 = _exact_sort16(hv, hi)
        return sv0, sv1, si0, si1

    _, _, si0, si1 = lax.fori_loop(
        0, nv, ph3, (inf16, inf16, zero16, zero16))

    obase = 20 * rw
    plsc.store_compressed(ob.at[pl.ds(obase, L)], si0, mask=_lane() >= 1)
    plsc.store_compressed(ob.at[pl.ds(obase + 15, L)], si1, mask=_lane() < 5)


def _process_chunk(xb, cand, ob, g):
    def row(r, _):
        _row_body(xb, cand, ob, r, CHUNK_ROWS * g + r)
        return 0
    lax.fori_loop(0, CHUNK_ROWS, row, 0)


_mesh = plsc.VectorSubcoreMesh(
    core_axis_name="c", subcore_axis_name="s", num_cores=NC, num_subcores=NS)


@functools.partial(
    pl.kernel,
    out_type=jax.ShapeDtypeStruct((B_ROWS * K,), jnp.int32),
    mesh=_mesh,
    scratch_types=[
        pltpu.VMEM((CHUNK_ELEMS,), jnp.float32),
        pltpu.VMEM((CHUNK_ELEMS,), jnp.float32),
        pltpu.VMEM((OUT_PER_W + 16,), jnp.int32),
        pltpu.VMEM((N + 16,), jnp.int32),
        pltpu.SemaphoreType.DMA,
        pltpu.SemaphoreType.DMA,
    ],
    compiler_params=pltpu.CompilerParams(needs_layout_passes=False),
)
def _sc_topk(x_hbm, o_hbm, xb0, xb1, ob, cand, sem0, sem1):
    wid = lax.axis_index("s") * NC + lax.axis_index("c")
    row0 = wid * ROWS_PER_W

    def chunk_src(g):
        gc = jnp.minimum(g, CHUNKS_PER_W - 1)
        return x_hbm.at[pl.ds((row0 + CHUNK_ROWS * gc) * N, CHUNK_ELEMS)]

    pltpu.async_copy(chunk_src(jnp.int32(0)), xb0, sem0)
    pltpu.async_copy(chunk_src(jnp.int32(1)), xb1, sem1)

    def pair(g2, _):
        g = 2 * g2
        pltpu.make_async_copy(chunk_src(g), xb0, sem0).wait()
        _process_chunk(xb0, cand, ob, g)
        pltpu.async_copy(chunk_src(g + 2), xb0, sem0)
        pltpu.make_async_copy(chunk_src(g + 1), xb1, sem1).wait()
        _process_chunk(xb1, cand, ob, g + 1)
        pltpu.async_copy(chunk_src(g + 3), xb1, sem1)
        return 0

    lax.fori_loop(0, CHUNKS_PER_W // 2, pair, 0)
    pltpu.make_async_copy(chunk_src(jnp.int32(0)), xb0, sem0).wait()
    pltpu.make_async_copy(chunk_src(jnp.int32(0)), xb1, sem1).wait()

    pltpu.sync_copy(ob.at[pl.ds(0, OUT_PER_W)],
                    o_hbm.at[pl.ds(wid * OUT_PER_W, OUT_PER_W)])


@jax.jit
def kernel(inputs):
    d = inputs
    b, q, n = d.shape
    flat = d.reshape(b * q * n)
    out = _sc_topk(flat)
    return out.reshape(b, q, K)


# speedup vs baseline: 1.6833x; 1.6833x over previous
import functools

import jax
import jax.numpy as jnp
from jax import lax
from jax.experimental import pallas as pl
from jax.experimental.pallas import tpu as pltpu
from jax.experimental.pallas import tpu_sc as plsc

K = 20
N = 2048
B_ROWS = 32768
NC, NS, L = 2, 16, 16
NW = NC * NS
ROWS_PER_W = B_ROWS // NW
CHUNK_ROWS = 8
CHUNKS_PER_W = ROWS_PER_W // CHUNK_ROWS
CHUNK_ELEMS = CHUNK_ROWS * N
OUT_PER_W = ROWS_PER_W * K

_INF = float("inf")


def _lane():
    return lax.iota(jnp.int32, L)


def _vsort(v):
    w, _ = plsc.sort_key_val(v, _lane())
    return w


def _exact_sort16(v, i):
    w, j = plsc.sort_key_val(v, i)
    occ, _ = plsc.scan_count(w)
    rs = _lane() - occ
    key2 = rs * jnp.int32(2048) + j
    _, j2 = plsc.sort_key_val(key2, j)
    return w, j2


def _lex_take(av, ai, bv, bi):
    return (av < bv) | ((av == bv) & (ai < bi))


def _merge32(a, b):
    rb = lax.rev(b, (0,))
    lo = jnp.minimum(a, rb)
    hi = jnp.maximum(a, rb)
    return _vsort(lo), _vsort(hi)


def _threshold(a0, a1, a2, a3):
    p0, p1 = _merge32(_vsort(a0), _vsort(a1))
    q0, q1 = _merge32(_vsort(a2), _vsort(a3))
    l0 = jnp.minimum(p0, lax.rev(q1, (0,)))
    l1 = jnp.minimum(p1, lax.rev(q0, (0,)))
    h = _vsort(jnp.maximum(l0, l1))
    return jnp.min(jnp.where(_lane() >= 4, h, _INF))


def _scalar(v16):
    return lax.squeeze(lax.slice(v16, (0,), (1,)), (0,))


def _pair_body(xb, cand0, cand1, ob, p, rw_base):
    baseA = N * (2 * p)
    baseB = baseA + N

    inf16 = jnp.full((L,), _INF)

    def ph1(i, acc):
        a = list(acc)
        o = baseA + 64 * i
        for q in range(4):
            a[q] = jnp.minimum(a[q], xb[pl.ds(o + 16 * q, L)])
            a[4 + q] = jnp.minimum(a[4 + q], xb[pl.ds(o + N + 16 * q, L)])
        return tuple(a)

    acc = lax.fori_loop(0, 32, ph1, (inf16,) * 8)
    tA = _threshold(*acc[:4])
    tB = _threshold(*acc[4:])
    tvA = jnp.full((L,), tA)
    tvB = jnp.full((L,), tB)

    def ph2(i, carry):
        cA, cB, idxv = carry
        o = baseA + 64 * i
        for q in range(4):
            vA = xb[pl.ds(o + 16 * q, L)]
            vB = xb[pl.ds(o + N + 16 * q, L)]
            mA = vA <= tvA
            mB = vB <= tvB
            iq = idxv + jnp.int32(16 * q)
            plsc.store_compressed(cand0.at[pl.ds(cA, L)], iq, mask=mA)
            plsc.store_compressed(cand1.at[pl.ds(cB, L)], iq, mask=mB)
            cA = cA + _scalar(plsc.all_reduce_population_count(mA))
            cB = cB + _scalar(plsc.all_reduce_population_count(mB))
        return cA, cB, idxv + jnp.int32(64)

    cA, cB, _ = lax.fori_loop(
        0, 32, ph2, (jnp.int32(0), jnp.int32(0), _lane()))

    zero16 = jnp.zeros((L,), jnp.int32)
    nv = (jnp.maximum(cA, cB) + jnp.int32(15)) >> 4

    def merge_step(sv0, sv1, si0, si1, cand, c, base, jv):
        lanepos = _lane() + 16 * jv
        m = lanepos < c
        idx = jnp.where(m, cand[pl.ds(16 * jv, L)], 0)
        vals = plsc.load_gather(xb.at[pl.ds(base, N)], [idx])
        vals = jnp.where(m, vals, _INF)
        w, j = _exact_sort16(vals, idx)
        rw_, rj_ = lax.rev(w, (0,)), lax.rev(j, (0,))
        take = _lex_take(sv1, si1, rw_, rj_)
        l1v = jnp.where(take, sv1, rw_)
        l1i = jnp.where(take, si1, rj_)
        take2 = _lex_take(sv0, si0, l1v, l1i)
        lv = jnp.where(take2, sv0, l1v)
        li = jnp.where(take2, si0, l1i)
        hv = jnp.where(take2, l1v, sv0)
        hi = jnp.where(take2, l1i, si0)
        sv0, si0 = _exact_sort16(lv, li)
        sv1, si1 = _exact_sort16(hv, hi)
        return sv0, sv1, si0, si1

    def ph3(jv, carry):
        sA = merge_step(*carry[0:2], *carry[4:6], cand0, cA, baseA, jv)
        sB = merge_step(*carry[2:4], *carry[6:8], cand1, cB, baseB, jv)
        return (sA[0], sA[1], sB[0], sB[1], sA[2], sA[3], sB[2], sB[3])

    init = (inf16, inf16, inf16, inf16, zero16, zero16, zero16, zero16)
    res = lax.fori_loop(0, nv, ph3, init)
    siA0, siA1, siB0, siB1 = res[4], res[5], res[6], res[7]

    oA = 20 * (rw_base + 2 * p)
    oB = oA + 20
    lo_mask = _lane() >= 1
    hi_mask = _lane() < 5
    plsc.store_compressed(ob.at[pl.ds(oA, L)], siA0, mask=lo_mask)
    plsc.store_compressed(ob.at[pl.ds(oA + 15, L)], siA1, mask=hi_mask)
    plsc.store_compressed(ob.at[pl.ds(oB, L)], siB0, mask=lo_mask)
    plsc.store_compressed(ob.at[pl.ds(oB + 15, L)], siB1, mask=hi_mask)


def _process_chunk(xb, cand0, cand1, ob, g):
    rw_base = CHUNK_ROWS * g

    def pairf(p, _):
        _pair_body(xb, cand0, cand1, ob, p, rw_base)
        return 0
    lax.fori_loop(0, CHUNK_ROWS // 2, pairf, 0)


_mesh = plsc.VectorSubcoreMesh(
    core_axis_name="c", subcore_axis_name="s", num_cores=NC, num_subcores=NS)


@functools.partial(
    pl.kernel,
    out_type=jax.ShapeDtypeStruct((B_ROWS * K,), jnp.int32),
    mesh=_mesh,
    scratch_types=[
        pltpu.VMEM((CHUNK_ELEMS,), jnp.float32),
        pltpu.VMEM((CHUNK_ELEMS,), jnp.float32),
        pltpu.VMEM((OUT_PER_W + 16,), jnp.int32),
        pltpu.VMEM((N + 16,), jnp.int32),
        pltpu.VMEM((N + 16,), jnp.int32),
        pltpu.SemaphoreType.DMA,
        pltpu.SemaphoreType.DMA,
    ],
    compiler_params=pltpu.CompilerParams(needs_layout_passes=False),
)
def _sc_topk(x_hbm, o_hbm, xb0, xb1, ob, cand0, cand1, sem0, sem1):
    wid = lax.axis_index("s") * NC + lax.axis_index("c")
    row0 = wid * ROWS_PER_W

    def chunk_src(g):
        gc = jnp.minimum(g, CHUNKS_PER_W - 1)
        return x_hbm.at[pl.ds((row0 + CHUNK_ROWS * gc) * N, CHUNK_ELEMS)]

    pltpu.async_copy(chunk_src(jnp.int32(0)), xb0, sem0)
    pltpu.async_copy(chunk_src(jnp.int32(1)), xb1, sem1)

    def pair(g2, _):
        g = 2 * g2
        pltpu.make_async_copy(chunk_src(g), xb0, sem0).wait()
        _process_chunk(xb0, cand0, cand1, ob, g)
        pltpu.async_copy(chunk_src(g + 2), xb0, sem0)
        pltpu.make_async_copy(chunk_src(g + 1), xb1, sem1).wait()
        _process_chunk(xb1, cand0, cand1, ob, g + 1)
        pltpu.async_copy(chunk_src(g + 3), xb1, sem1)
        return 0

    lax.fori_loop(0, CHUNKS_PER_W // 2, pair, 0)
    pltpu.make_async_copy(chunk_src(jnp.int32(0)), xb0, sem0).wait()
    pltpu.make_async_copy(chunk_src(jnp.int32(0)), xb1, sem1).wait()

    pltpu.sync_copy(ob.at[pl.ds(0, OUT_PER_W)],
                    o_hbm.at[pl.ds(wid * OUT_PER_W, OUT_PER_W)])


@jax.jit
def kernel(inputs):
    d = inputs
    b, q, n = d.shape
    flat = d.reshape(b * q * n)
    out = _sc_topk(flat)
    return out.reshape(b, q, K)
